# baseline (speedup 1.0000x reference)
import jax
import jax.numpy as jnp
from jax.experimental import pallas as pl
from jax.experimental.pallas import tpu as pltpu

_VB = 8192


def _mask_where_kernel(mask_ref, x_ref, o_ref):
    m = mask_ref[0, :] != 0
    o_ref[...] = jnp.where(m[None, :], -jnp.inf, x_ref[...])


def kernel(logits, generated_so_far, forbidden_token_mask):
    del generated_so_far
    B, V = logits.shape
    mask2d = forbidden_token_mask.astype(jnp.int8).reshape(1, V)
    return pl.pallas_call(
        _mask_where_kernel,
        grid=(pl.cdiv(V, _VB),),
        in_specs=[
            pl.BlockSpec((1, _VB), lambda i: (0, i)),
            pl.BlockSpec((B, _VB), lambda i: (0, i)),
        ],
        out_specs=pl.BlockSpec((B, _VB), lambda i: (0, i)),
        out_shape=jax.ShapeDtypeStruct((B, V), logits.dtype),
        compiler_params=pltpu.CompilerParams(
            dimension_semantics=("parallel",)),
    )(mask2d, logits)


# speedup vs baseline: 1.2736x; 1.2736x over previous
import jax
import jax.numpy as jnp
from jax.experimental import pallas as pl
from jax.experimental.pallas import tpu as pltpu

_VB = 8192


def _mask_where_kernel(mask_ref, x_ref, o_ref):
    m8 = jnp.broadcast_to(mask_ref[0:1, :] != 0, (8, _VB))
    neg_inf = jnp.full((8, _VB), -jnp.inf, dtype=o_ref.dtype)
    for r in range(0, x_ref.shape[0], 8):
        o_ref[r:r + 8, :] = jnp.where(m8, neg_inf, x_ref[r:r + 8, :])


def kernel(logits, generated_so_far, forbidden_token_mask):
    del generated_so_far
    B, V = logits.shape
    mask2d = forbidden_token_mask.astype(jnp.int8).reshape(1, V)
    return pl.pallas_call(
        _mask_where_kernel,
        grid=(pl.cdiv(V, _VB),),
        in_specs=[
            pl.BlockSpec((1, _VB), lambda i: (0, i)),
            pl.BlockSpec((B, _VB), lambda i: (0, i)),
        ],
        out_specs=pl.BlockSpec((B, _VB), lambda i: (0, i)),
        out_shape=jax.ShapeDtypeStruct((B, V), logits.dtype),
        compiler_params=pltpu.CompilerParams(
            dimension_semantics=("parallel",)),
    )(mask2d, logits)
